# baseline (speedup 1.0000x reference)
import functools

import jax
import jax.numpy as jnp
from jax import lax
from jax.experimental import pallas as pl
from jax.experimental.pallas import tpu as pltpu
from jax.experimental.pallas import tpu_sc as plsc

_N = 5000
_L = 16
_NW = 32
_RPW = 160
_NPAD = _NW * _RPW
_NCHUNK = _NPAD // _L


def _nms_body(xl_h, yl_h, xh_h, yh_h, sc_h, lab_h, out_h,
              xl, yl, xh, yh, sc, lab, outv):
    cid = lax.axis_index("c")
    sid = lax.axis_index("s")
    wid = sid * 2 + cid
    pltpu.sync_copy(xl_h, xl)
    pltpu.sync_copy(yl_h, yl)
    pltpu.sync_copy(xh_h, xh)
    pltpu.sync_copy(yh_h, yh)
    pltpu.sync_copy(sc_h, sc)
    pltpu.sync_copy(lab_h, lab)
    base = wid * _RPW
    lane = lax.iota(jnp.int32, _L)
    lane0 = lane == 0

    def row_body(r, carry_row):
        i = base + r
        isp = jnp.full((_L,), i, dtype=jnp.int32)
        xli = plsc.load_gather(xl, [isp])
        yli = plsc.load_gather(yl, [isp])
        xhi = plsc.load_gather(xh, [isp])
        yhi = plsc.load_gather(yh, [isp])
        sci = plsc.load_gather(sc, [isp])
        labi = plsc.load_gather(lab, [isp])
        areai = (xhi - xli) * (yhi - yli)

        def chunk_body(cidx, carry):
            acc, jv = carry
            off = cidx * _L
            xlj = xl[pl.ds(off, _L)]
            ylj = yl[pl.ds(off, _L)]
            xhj = xh[pl.ds(off, _L)]
            yhj = yh[pl.ds(off, _L)]
            scj = sc[pl.ds(off, _L)]
            labj = lab[pl.ds(off, _L)]
            w = jnp.maximum(jnp.minimum(xhi, xhj) - jnp.maximum(xli, xlj), 0.0)
            h = jnp.maximum(jnp.minimum(yhi, yhj) - jnp.maximum(yli, ylj), 0.0)
            inter = w * h
            areaj = (xhj - xlj) * (yhj - ylj)
            union = areai + areaj - inter
            iou_hit = inter > union * 0.5
            same = labi == labj
            pri = (scj > sci) | ((scj == sci) & (jv > isp))
            return acc | (iou_hit & same & pri), jv + _L

        acc0 = jnp.zeros((_L,), dtype=jnp.bool_)
        acc, _ = lax.fori_loop(0, _NCHUNK, chunk_body, (acc0, lane))
        supp = jnp.full((_L,), jnp.any(acc))
        outvec = jnp.where(supp, 0.0, sci)
        plsc.store_scatter(outv, [jnp.full((_L,), r, dtype=jnp.int32)],
                           outvec, mask=lane0)
        return carry_row

    lax.fori_loop(0, _RPW, row_body, 0)
    pltpu.sync_copy(outv, out_h.at[pl.ds(base, _RPW)])


_nms = functools.partial(
    pl.kernel,
    out_type=jax.ShapeDtypeStruct((_NPAD,), jnp.float32),
    mesh=plsc.VectorSubcoreMesh(core_axis_name="c", subcore_axis_name="s"),
    compiler_params=pltpu.CompilerParams(needs_layout_passes=False),
    scratch_types=[
        pltpu.VMEM((_NPAD,), jnp.float32),
        pltpu.VMEM((_NPAD,), jnp.float32),
        pltpu.VMEM((_NPAD,), jnp.float32),
        pltpu.VMEM((_NPAD,), jnp.float32),
        pltpu.VMEM((_NPAD,), jnp.float32),
        pltpu.VMEM((_NPAD,), jnp.int32),
        pltpu.VMEM((_RPW,), jnp.float32),
    ],
)(_nms_body)


def kernel(boxes, scores, pred_labels):
    padf = jnp.zeros((_NPAD - _N,), jnp.float32)
    xl = jnp.concatenate([boxes[:, 0], padf])
    yl = jnp.concatenate([boxes[:, 1], padf])
    xh = jnp.concatenate([boxes[:, 2], padf])
    yh = jnp.concatenate([boxes[:, 3], padf])
    sc = jnp.concatenate([scores, padf])
    lab = jnp.concatenate([pred_labels.astype(jnp.int32),
                           jnp.full((_NPAD - _N,), -1, jnp.int32)])
    out = _nms(xl, yl, xh, yh, sc, lab)
    return out[:_N]


# speedup vs baseline: 2.1163x; 2.1163x over previous
import functools

import jax
import jax.numpy as jnp
from jax import lax
from jax.experimental import pallas as pl
from jax.experimental.pallas import tpu as pltpu
from jax.experimental.pallas import tpu_sc as plsc

_N = 5000
_L = 16
_NW = 32
_RPW = 160
_NPAD = _NW * _RPW
_CPW = _RPW // _L

_BIG = 1e30


def _nms_body(xl_h, yl_h, xh_h, yh_h, sc_h, lab_h, ord_h, lo_h, hi_h, out_h,
              xl, yl, xh, yh, sc, lab, ordv,
              xs, ys, xe, ye, ss, lb,
              lov, hiv, outv):
    cid = lax.axis_index("c")
    sid = lax.axis_index("s")
    wid = sid * 2 + cid
    pltpu.sync_copy(xl_h, xl)
    pltpu.sync_copy(yl_h, yl)
    pltpu.sync_copy(xh_h, xh)
    pltpu.sync_copy(yh_h, yh)
    pltpu.sync_copy(sc_h, sc)
    pltpu.sync_copy(lab_h, lab)
    pltpu.sync_copy(ord_h, ordv)
    pltpu.sync_copy(lo_h, lov)
    pltpu.sync_copy(hi_h, hiv)

    lane = lax.iota(jnp.int32, _L)
    lane0 = lane == 0
    wsp = jnp.full((_L,), wid, dtype=jnp.int32)
    lo = jnp.max(plsc.load_gather(lov, [wsp]))
    hi = jnp.max(plsc.load_gather(hiv, [wsp]))
    clo = lo >> 4
    chi = (hi + (_L - 1)) >> 4
    base = wid * _RPW
    plo = jnp.minimum(clo, wid * _CPW)
    phi = jnp.maximum(chi, wid * _CPW + _CPW)

    def perm_body(c, carry):
        off = c * _L
        idxv = ordv[pl.ds(off, _L)]
        xs[pl.ds(off, _L)] = plsc.load_gather(xl, [idxv])
        ys[pl.ds(off, _L)] = plsc.load_gather(yl, [idxv])
        xe[pl.ds(off, _L)] = plsc.load_gather(xh, [idxv])
        ye[pl.ds(off, _L)] = plsc.load_gather(yh, [idxv])
        ss[pl.ds(off, _L)] = plsc.load_gather(sc, [idxv])
        lb[pl.ds(off, _L)] = plsc.load_gather(lab, [idxv])
        return carry

    lax.fori_loop(plo, phi, perm_body, 0)

    def row_body(r, carry_row):
        i = base + r
        isp = jnp.full((_L,), i, dtype=jnp.int32)
        xli = plsc.load_gather(xs, [isp])
        yli = plsc.load_gather(ys, [isp])
        xhi = plsc.load_gather(xe, [isp])
        yhi = plsc.load_gather(ye, [isp])
        sci = plsc.load_gather(ss, [isp])
        labi = plsc.load_gather(lb, [isp])
        ogi = plsc.load_gather(ordv, [isp])
        areai = (xhi - xli) * (yhi - yli)

        def chunk_body(cidx, acc):
            off = cidx * _L
            xlj = xs[pl.ds(off, _L)]
            ylj = ys[pl.ds(off, _L)]
            xhj = xe[pl.ds(off, _L)]
            yhj = ye[pl.ds(off, _L)]
            scj = ss[pl.ds(off, _L)]
            labj = lb[pl.ds(off, _L)]
            ogj = ordv[pl.ds(off, _L)]
            w = jnp.maximum(jnp.minimum(xhi, xhj) - jnp.maximum(xli, xlj), 0.0)
            h = jnp.maximum(jnp.minimum(yhi, yhj) - jnp.maximum(yli, ylj), 0.0)
            inter = w * h
            areaj = (xhj - xlj) * (yhj - ylj)
            union = areai + areaj - inter
            iou_hit = inter > union * 0.5
            same = labi == labj
            pri = (scj > sci) | ((scj == sci) & (ogj > ogi))
            return acc | (iou_hit & same & pri)

        acc0 = jnp.zeros((_L,), dtype=jnp.bool_)
        acc = lax.fori_loop(clo, chi, chunk_body, acc0)
        supp = jnp.full((_L,), jnp.any(acc))
        outvec = jnp.where(supp, 0.0, sci)
        plsc.store_scatter(outv, [jnp.full((_L,), r, dtype=jnp.int32)],
                           outvec, mask=lane0)
        return carry_row

    lax.fori_loop(0, _RPW, row_body, 0)
    pltpu.sync_copy(outv, out_h.at[pl.ds(base, _RPW)])


_nms = functools.partial(
    pl.kernel,
    out_type=jax.ShapeDtypeStruct((_NPAD,), jnp.float32),
    mesh=plsc.VectorSubcoreMesh(core_axis_name="c", subcore_axis_name="s"),
    compiler_params=pltpu.CompilerParams(needs_layout_passes=False),
    scratch_types=[
        pltpu.VMEM((_NPAD,), jnp.float32),
        pltpu.VMEM((_NPAD,), jnp.float32),
        pltpu.VMEM((_NPAD,), jnp.float32),
        pltpu.VMEM((_NPAD,), jnp.float32),
        pltpu.VMEM((_NPAD,), jnp.float32),
        pltpu.VMEM((_NPAD,), jnp.int32),
        pltpu.VMEM((_NPAD,), jnp.int32),
        pltpu.VMEM((_NPAD,), jnp.float32),
        pltpu.VMEM((_NPAD,), jnp.float32),
        pltpu.VMEM((_NPAD,), jnp.float32),
        pltpu.VMEM((_NPAD,), jnp.float32),
        pltpu.VMEM((_NPAD,), jnp.float32),
        pltpu.VMEM((_NPAD,), jnp.int32),
        pltpu.VMEM((_NW,), jnp.int32),
        pltpu.VMEM((_NW,), jnp.int32),
        pltpu.VMEM((_RPW,), jnp.float32),
    ],
)(_nms_body)


def kernel(boxes, scores, pred_labels):
    xl = boxes[:, 0]
    yl = boxes[:, 1]
    xh = boxes[:, 2]
    yh = boxes[:, 3]
    wmax = jnp.max(xh - xl)

    iota = jnp.arange(_N, dtype=jnp.int32)
    xls, order = lax.sort((xl, iota), num_keys=1)

    firsts = xls[0::_RPW]
    lasts = jnp.concatenate([xls[_RPW - 1::_RPW], xls[_N - 1:]])
    lo_arr = jnp.searchsorted(xls, firsts - wmax, side="left").astype(jnp.int32)
    hi_arr = jnp.searchsorted(xls, lasts + wmax, side="right").astype(jnp.int32)

    npadf = jnp.zeros((_NPAD - _N,), jnp.float32)
    xl_p = jnp.concatenate([xl, jnp.full((_NPAD - _N,), _BIG, jnp.float32)])
    yl_p = jnp.concatenate([yl, npadf])
    xh_p = jnp.concatenate([xh, npadf])
    yh_p = jnp.concatenate([yh, npadf])
    sc_p = jnp.concatenate([scores, npadf])
    lab_p = jnp.concatenate([pred_labels.astype(jnp.int32),
                             jnp.full((_NPAD - _N,), -1, jnp.int32)])
    ord_p = jnp.concatenate([order,
                             jnp.arange(_N, _NPAD, dtype=jnp.int32)])

    out_sorted = _nms(xl_p, yl_p, xh_p, yh_p, sc_p, lab_p,
                      ord_p, lo_arr, hi_arr)
    return jnp.zeros((_N,), jnp.float32).at[order].set(
        out_sorted[:_N], unique_indices=True)


# speedup vs baseline: 2.2466x; 1.0616x over previous
import functools

import jax
import jax.numpy as jnp
from jax import lax
from jax.experimental import pallas as pl
from jax.experimental.pallas import tpu as pltpu
from jax.experimental.pallas import tpu_sc as plsc

_N = 5000
_L = 16
_NW = 32
_RPW = 160
_NPAD = _NW * _RPW
_CPW = _RPW // _L

_BIG = 1e30
_OB = 80
_OR = _RPW // _OB


def _nms_body(xl_h, yl_h, xh_h, yh_h, sc_h, lab_h, ord_h, ord3_h, lo_h, hi_h,
              out_h,
              xl, yl, xh, yh, sc, lab, ordv,
              xs, ys, xe, ye, ss, lb,
              lov, hiv, outv, myidx_a, myidx_b, sem):
    cid = lax.axis_index("c")
    sid = lax.axis_index("s")
    wid = sid * 2 + cid
    pltpu.sync_copy(xl_h, xl)
    pltpu.sync_copy(yl_h, yl)
    pltpu.sync_copy(xh_h, xh)
    pltpu.sync_copy(yh_h, yh)
    pltpu.sync_copy(sc_h, sc)
    pltpu.sync_copy(lab_h, lab)
    pltpu.sync_copy(ord_h, ordv)
    pltpu.sync_copy(lo_h, lov)
    pltpu.sync_copy(hi_h, hiv)

    lane = lax.iota(jnp.int32, _L)
    lane0 = lane == 0
    wsp = jnp.full((_L,), wid, dtype=jnp.int32)
    lo = jnp.max(plsc.load_gather(lov, [wsp]))
    hi = jnp.max(plsc.load_gather(hiv, [wsp]))
    clo = lo >> 4
    chi = (hi + (_L - 1)) >> 4
    base = wid * _RPW
    plo = jnp.minimum(clo, wid * _CPW)
    phi = jnp.maximum(chi, wid * _CPW + _CPW)

    def perm_body(c, carry):
        off = c * _L
        idxv = ordv[pl.ds(off, _L)]
        xs[pl.ds(off, _L)] = plsc.load_gather(xl, [idxv])
        ys[pl.ds(off, _L)] = plsc.load_gather(yl, [idxv])
        xe[pl.ds(off, _L)] = plsc.load_gather(xh, [idxv])
        ye[pl.ds(off, _L)] = plsc.load_gather(yh, [idxv])
        ss[pl.ds(off, _L)] = plsc.load_gather(sc, [idxv])
        lb[pl.ds(off, _L)] = plsc.load_gather(lab, [idxv])
        return carry

    lax.fori_loop(plo, phi, perm_body, 0)

    def row_body(r, carry_row):
        i = base + r
        isp = jnp.full((_L,), i, dtype=jnp.int32)
        xli = plsc.load_gather(xs, [isp])
        yli = plsc.load_gather(ys, [isp])
        xhi = plsc.load_gather(xe, [isp])
        yhi = plsc.load_gather(ye, [isp])
        sci = plsc.load_gather(ss, [isp])
        labi = plsc.load_gather(lb, [isp])
        ogi = plsc.load_gather(ordv, [isp])
        areai = (xhi - xli) * (yhi - yli)

        def chunk_body(cidx, acc):
            off = cidx * _L
            xlj = xs[pl.ds(off, _L)]
            ylj = ys[pl.ds(off, _L)]
            xhj = xe[pl.ds(off, _L)]
            yhj = ye[pl.ds(off, _L)]
            scj = ss[pl.ds(off, _L)]
            labj = lb[pl.ds(off, _L)]
            ogj = ordv[pl.ds(off, _L)]
            w = jnp.maximum(jnp.minimum(xhi, xhj) - jnp.maximum(xli, xlj), 0.0)
            h = jnp.maximum(jnp.minimum(yhi, yhj) - jnp.maximum(yli, ylj), 0.0)
            inter = w * h
            areaj = (xhj - xlj) * (yhj - ylj)
            union = areai + areaj - inter
            iou_hit = inter > union * 0.5
            same = labi == labj
            pri = (scj > sci) | ((scj == sci) & (ogj > ogi))
            return acc | (iou_hit & same & pri)

        acc0 = jnp.zeros((_L,), dtype=jnp.bool_)
        acc = lax.fori_loop(clo, chi, chunk_body, acc0)
        supp = jnp.full((_L,), jnp.any(acc))
        outvec = jnp.where(supp, 0.0, sci)
        plsc.store_scatter(outv, [jnp.full((_L,), r, dtype=jnp.int32)],
                           outvec, mask=lane0)
        return carry_row

    lax.fori_loop(0, _RPW, row_body, 0)
    pltpu.sync_copy(ord3_h.at[wid, 0], myidx_a)
    pltpu.sync_copy(ord3_h.at[wid, 1], myidx_b)
    pltpu.async_copy(outv.at[pl.ds(0, _OB)], out_h.at[myidx_a], sem).wait()
    pltpu.async_copy(outv.at[pl.ds(_OB, _OB)], out_h.at[myidx_b], sem).wait()


_nms = functools.partial(
    pl.kernel,
    out_type=jax.ShapeDtypeStruct((_NPAD,), jnp.float32),
    mesh=plsc.VectorSubcoreMesh(core_axis_name="c", subcore_axis_name="s"),
    compiler_params=pltpu.CompilerParams(needs_layout_passes=False),
    scratch_types=[
        pltpu.VMEM((_NPAD,), jnp.float32),
        pltpu.VMEM((_NPAD,), jnp.float32),
        pltpu.VMEM((_NPAD,), jnp.float32),
        pltpu.VMEM((_NPAD,), jnp.float32),
        pltpu.VMEM((_NPAD,), jnp.float32),
        pltpu.VMEM((_NPAD,), jnp.int32),
        pltpu.VMEM((_NPAD,), jnp.int32),
        pltpu.VMEM((_NPAD,), jnp.float32),
        pltpu.VMEM((_NPAD,), jnp.float32),
        pltpu.VMEM((_NPAD,), jnp.float32),
        pltpu.VMEM((_NPAD,), jnp.float32),
        pltpu.VMEM((_NPAD,), jnp.float32),
        pltpu.VMEM((_NPAD,), jnp.int32),
        pltpu.VMEM((_NW,), jnp.int32),
        pltpu.VMEM((_NW,), jnp.int32),
        pltpu.VMEM((_RPW,), jnp.float32),
        pltpu.VMEM((_OB,), jnp.int32),
        pltpu.VMEM((_OB,), jnp.int32),
        pltpu.SemaphoreType.DMA,
    ],
)(_nms_body)


def kernel(boxes, scores, pred_labels):
    xl = boxes[:, 0]
    yl = boxes[:, 1]
    xh = boxes[:, 2]
    yh = boxes[:, 3]
    wmax = jnp.max(xh - xl)

    iota = jnp.arange(_N, dtype=jnp.int32)
    xls, order = lax.sort((xl, iota), num_keys=1)

    firsts = xls[0::_RPW]
    lasts = jnp.concatenate([xls[_RPW - 1::_RPW], xls[_N - 1:]])
    lo_arr = jnp.searchsorted(xls, firsts - wmax, side="left").astype(jnp.int32)
    hi_arr = jnp.searchsorted(xls, lasts + wmax, side="right").astype(jnp.int32)

    npadf = jnp.zeros((_NPAD - _N,), jnp.float32)
    xl_p = jnp.concatenate([xl, jnp.full((_NPAD - _N,), _BIG, jnp.float32)])
    yl_p = jnp.concatenate([yl, npadf])
    xh_p = jnp.concatenate([xh, npadf])
    yh_p = jnp.concatenate([yh, npadf])
    sc_p = jnp.concatenate([scores, npadf])
    lab_p = jnp.concatenate([pred_labels.astype(jnp.int32),
                             jnp.full((_NPAD - _N,), -1, jnp.int32)])
    ord_p = jnp.concatenate([order,
                             jnp.arange(_N, _NPAD, dtype=jnp.int32)])

    out = _nms(xl_p, yl_p, xh_p, yh_p, sc_p, lab_p,
               ord_p, ord_p.reshape(_NW, _OR, _OB), lo_arr, hi_arr)
    return out[:_N]


# speedup vs baseline: 2.2666x; 1.0089x over previous
import functools

import jax
import jax.numpy as jnp
from jax import lax
from jax.experimental import pallas as pl
from jax.experimental.pallas import tpu as pltpu
from jax.experimental.pallas import tpu_sc as plsc

_N = 5000
_L = 16
_NW = 32
_RPW = 160
_NPAD = _NW * _RPW
_CPW = _RPW // _L

_BIG = 1e30
_OB = 80
_OR = _RPW // _OB


def _nms_body(xl_h, yl_h, xh_h, yh_h, sc_h, lab_h, ord_h, ord3_h, lo_h, hi_h,
              out_h,
              xl, yl, xh, yh, sc, lab, ordv,
              xs, ys, xe, ye, ss, lb,
              lov, hiv, outv, myidx_a, myidx_b, sem):
    cid = lax.axis_index("c")
    sid = lax.axis_index("s")
    wid = sid * 2 + cid
    pltpu.sync_copy(xl_h, xl)
    pltpu.sync_copy(yl_h, yl)
    pltpu.sync_copy(xh_h, xh)
    pltpu.sync_copy(yh_h, yh)
    pltpu.sync_copy(sc_h, sc)
    pltpu.sync_copy(lab_h, lab)
    pltpu.sync_copy(ord_h, ordv)
    pltpu.sync_copy(lo_h, lov)
    pltpu.sync_copy(hi_h, hiv)

    lane = lax.iota(jnp.int32, _L)
    lane0 = lane == 0
    wsp = jnp.full((_L,), wid, dtype=jnp.int32)
    lo = jnp.max(plsc.load_gather(lov, [wsp]))
    hi = jnp.max(plsc.load_gather(hiv, [wsp]))
    clo = lo >> 4
    chi = (hi + (_L - 1)) >> 4
    base = wid * _RPW
    plo = jnp.minimum(clo, wid * _CPW)
    phi = jnp.maximum(chi, wid * _CPW + _CPW)

    def perm_body(c, carry):
        off = c * _L
        idxv = ordv[pl.ds(off, _L)]
        xs[pl.ds(off, _L)] = plsc.load_gather(xl, [idxv])
        ys[pl.ds(off, _L)] = plsc.load_gather(yl, [idxv])
        xe[pl.ds(off, _L)] = plsc.load_gather(xh, [idxv])
        ye[pl.ds(off, _L)] = plsc.load_gather(yh, [idxv])
        ss[pl.ds(off, _L)] = plsc.load_gather(sc, [idxv])
        lb[pl.ds(off, _L)] = plsc.load_gather(lab, [idxv])
        return carry

    lax.fori_loop(plo, phi, perm_body, 0)

    def row_body(r, carry_row):
        i = base + r
        isp = jnp.full((_L,), i, dtype=jnp.int32)
        xli = plsc.load_gather(xs, [isp])
        yli = plsc.load_gather(ys, [isp])
        xhi = plsc.load_gather(xe, [isp])
        yhi = plsc.load_gather(ye, [isp])
        sci = plsc.load_gather(ss, [isp])
        labi = plsc.load_gather(lb, [isp])
        ogi = plsc.load_gather(ordv, [isp])
        areai = (xhi - xli) * (yhi - yli)

        def chunk_body(cidx, acc):
            off = cidx * _L
            xlj = xs[pl.ds(off, _L)]
            ylj = ys[pl.ds(off, _L)]
            xhj = xe[pl.ds(off, _L)]
            yhj = ye[pl.ds(off, _L)]
            scj = ss[pl.ds(off, _L)]
            labj = lb[pl.ds(off, _L)]
            ogj = ordv[pl.ds(off, _L)]
            w = jnp.maximum(jnp.minimum(xhi, xhj) - jnp.maximum(xli, xlj), 0.0)
            h = jnp.maximum(jnp.minimum(yhi, yhj) - jnp.maximum(yli, ylj), 0.0)
            inter = w * h
            areaj = (xhj - xlj) * (yhj - ylj)
            union = areai + areaj - inter
            iou_hit = inter > union * 0.5
            same = labi == labj
            pri = (scj > sci) | ((scj == sci) & (ogj > ogi))
            return acc | (iou_hit & same & pri)

        acc0 = jnp.zeros((_L,), dtype=jnp.bool_)
        acc = lax.fori_loop(clo, chi, chunk_body, acc0)
        supp = jnp.full((_L,), jnp.any(acc))
        outvec = jnp.where(supp, 0.0, sci)
        plsc.store_scatter(outv, [jnp.full((_L,), r, dtype=jnp.int32)],
                           outvec, mask=lane0)
        return carry_row

    lax.fori_loop(0, _RPW, row_body, 0)
    pltpu.sync_copy(ord3_h.at[wid, 0], myidx_a)
    pltpu.sync_copy(ord3_h.at[wid, 1], myidx_b)
    pltpu.async_copy(outv.at[pl.ds(0, _OB)], out_h.at[myidx_a], sem).wait()
    pltpu.async_copy(outv.at[pl.ds(_OB, _OB)], out_h.at[myidx_b], sem).wait()


_nms = functools.partial(
    pl.kernel,
    out_type=jax.ShapeDtypeStruct((_NPAD,), jnp.float32),
    mesh=plsc.VectorSubcoreMesh(core_axis_name="c", subcore_axis_name="s"),
    compiler_params=pltpu.CompilerParams(needs_layout_passes=False),
    scratch_types=[
        pltpu.VMEM((_NPAD,), jnp.float32),
        pltpu.VMEM((_NPAD,), jnp.float32),
        pltpu.VMEM((_NPAD,), jnp.float32),
        pltpu.VMEM((_NPAD,), jnp.float32),
        pltpu.VMEM((_NPAD,), jnp.float32),
        pltpu.VMEM((_NPAD,), jnp.int32),
        pltpu.VMEM((_NPAD,), jnp.int32),
        pltpu.VMEM((_NPAD,), jnp.float32),
        pltpu.VMEM((_NPAD,), jnp.float32),
        pltpu.VMEM((_NPAD,), jnp.float32),
        pltpu.VMEM((_NPAD,), jnp.float32),
        pltpu.VMEM((_NPAD,), jnp.float32),
        pltpu.VMEM((_NPAD,), jnp.int32),
        pltpu.VMEM((_NW,), jnp.int32),
        pltpu.VMEM((_NW,), jnp.int32),
        pltpu.VMEM((_RPW,), jnp.float32),
        pltpu.VMEM((_OB,), jnp.int32),
        pltpu.VMEM((_OB,), jnp.int32),
        pltpu.SemaphoreType.DMA,
    ],
)(_nms_body)


def kernel(boxes, scores, pred_labels):
    xl = boxes[:, 0]
    yl = boxes[:, 1]
    xh = boxes[:, 2]
    yh = boxes[:, 3]
    wmax = jnp.max(xh - xl)

    iota = jnp.arange(_N, dtype=jnp.int32)
    xls, order = lax.sort((xl, iota), num_keys=1, is_stable=False)

    firsts = xls[0::_RPW]
    lasts = jnp.concatenate([xls[_RPW - 1::_RPW], xls[_N - 1:]])
    lo_arr = jnp.searchsorted(xls, firsts - wmax, side="left").astype(jnp.int32)
    hi_arr = jnp.searchsorted(xls, lasts + wmax, side="right").astype(jnp.int32)

    npadf = jnp.zeros((_NPAD - _N,), jnp.float32)
    xl_p = jnp.concatenate([xl, jnp.full((_NPAD - _N,), _BIG, jnp.float32)])
    yl_p = jnp.concatenate([yl, npadf])
    xh_p = jnp.concatenate([xh, npadf])
    yh_p = jnp.concatenate([yh, npadf])
    sc_p = jnp.concatenate([scores, npadf])
    lab_p = jnp.concatenate([pred_labels.astype(jnp.int32),
                             jnp.full((_NPAD - _N,), -1, jnp.int32)])
    ord_p = jnp.concatenate([order,
                             jnp.arange(_N, _NPAD, dtype=jnp.int32)])

    out = _nms(xl_p, yl_p, xh_p, yh_p, sc_p, lab_p,
               ord_p, ord_p.reshape(_NW, _OR, _OB), lo_arr, hi_arr)
    return out[:_N]


# speedup vs baseline: 2.4072x; 1.0620x over previous
import functools

import jax
import jax.numpy as jnp
from jax import lax
from jax.experimental import pallas as pl
from jax.experimental.pallas import tpu as pltpu
from jax.experimental.pallas import tpu_sc as plsc

_N = 5000
_L = 16
_NW = 32
_RPW = 160
_NPAD = _NW * _RPW
_CPW = _RPW // _L

_BIG = 1e30
_OB = 80
_OR = _RPW // _OB


def _nms_body(xl_h, yl_h, xh_h, yh_h, sc_h, lab_h, ord_h, ord3_h, lo_h, hi_h,
              out_h,
              xl, yl, xh, yh, sc, lab, ordv,
              xs, ys, xe, ye, ss, po, ar,
              lov, hiv, outv, myidx_a, myidx_b, sem):
    cid = lax.axis_index("c")
    sid = lax.axis_index("s")
    wid = sid * 2 + cid
    pltpu.sync_copy(xl_h, xl)
    pltpu.sync_copy(yl_h, yl)
    pltpu.sync_copy(xh_h, xh)
    pltpu.sync_copy(yh_h, yh)
    pltpu.sync_copy(sc_h, sc)
    pltpu.sync_copy(lab_h, lab)
    pltpu.sync_copy(ord_h, ordv)
    pltpu.sync_copy(lo_h, lov)
    pltpu.sync_copy(hi_h, hiv)

    lane = lax.iota(jnp.int32, _L)
    lane0 = lane == 0
    wsp = jnp.full((_L,), wid, dtype=jnp.int32)
    lo = jnp.max(plsc.load_gather(lov, [wsp]))
    hi = jnp.max(plsc.load_gather(hiv, [wsp]))
    clo = lo >> 4
    chi = (hi + (_L - 1)) >> 4
    base = wid * _RPW
    plo = jnp.minimum(clo, wid * _CPW)
    phi = jnp.maximum(chi, wid * _CPW + _CPW)

    def perm_body(c, carry):
        off = c * _L
        idxv = ordv[pl.ds(off, _L)]
        xsg = plsc.load_gather(xl, [idxv])
        ysg = plsc.load_gather(yl, [idxv])
        xeg = plsc.load_gather(xh, [idxv])
        yeg = plsc.load_gather(yh, [idxv])
        xs[pl.ds(off, _L)] = xsg
        ys[pl.ds(off, _L)] = ysg
        xe[pl.ds(off, _L)] = xeg
        ye[pl.ds(off, _L)] = yeg
        ss[pl.ds(off, _L)] = plsc.load_gather(sc, [idxv])
        labg = plsc.load_gather(lab, [idxv])
        po[pl.ds(off, _L)] = (idxv << 3) | (labg & 7)
        ar[pl.ds(off, _L)] = (xeg - xsg) * (yeg - ysg)
        return carry

    lax.fori_loop(plo, phi, perm_body, 0)

    def _row_bcast(i):
        isp = jnp.full((_L,), i, dtype=jnp.int32)
        return (plsc.load_gather(xs, [isp]), plsc.load_gather(ys, [isp]),
                plsc.load_gather(xe, [isp]), plsc.load_gather(ye, [isp]),
                plsc.load_gather(ss, [isp]), plsc.load_gather(po, [isp]),
                plsc.load_gather(ar, [isp]))

    def row_body(r2, carry_row):
        r0 = r2 * 2
        r1 = r0 + 1
        b0 = _row_bcast(base + r0)
        b1 = _row_bcast(base + r1)

        def chunk_body(cidx, accs):
            acc0, acc1 = accs
            off = cidx * _L
            xlj = xs[pl.ds(off, _L)]
            ylj = ys[pl.ds(off, _L)]
            xhj = xe[pl.ds(off, _L)]
            yhj = ye[pl.ds(off, _L)]
            scj = ss[pl.ds(off, _L)]
            poj = po[pl.ds(off, _L)]
            arj = ar[pl.ds(off, _L)]

            def one(b, acc):
                xli, yli, xhi, yhi, sci, poi, ari = b
                w = jnp.maximum(
                    jnp.minimum(xhi, xhj) - jnp.maximum(xli, xlj), 0.0)
                h = jnp.maximum(
                    jnp.minimum(yhi, yhj) - jnp.maximum(yli, ylj), 0.0)
                inter = w * h
                union = ari + arj - inter
                iou_hit = inter + inter > union
                same = ((poj ^ poi) & 7) == 0
                pri = (scj > sci) | ((scj == sci) & (poj > poi))
                return acc | (iou_hit & same & pri)

            return one(b0, acc0), one(b1, acc1)

        acc0 = jnp.zeros((_L,), dtype=jnp.bool_)
        acc0, acc1 = lax.fori_loop(clo, chi, chunk_body, (acc0, acc0))
        out0 = jnp.where(jnp.full((_L,), jnp.any(acc0)), 0.0, b0[4])
        out1 = jnp.where(jnp.full((_L,), jnp.any(acc1)), 0.0, b1[4])
        plsc.store_scatter(outv, [jnp.full((_L,), r0, dtype=jnp.int32)],
                           out0, mask=lane0)
        plsc.store_scatter(outv, [jnp.full((_L,), r1, dtype=jnp.int32)],
                           out1, mask=lane0)
        return carry_row

    lax.fori_loop(0, _RPW // 2, row_body, 0)
    pltpu.sync_copy(ord3_h.at[wid, 0], myidx_a)
    pltpu.sync_copy(ord3_h.at[wid, 1], myidx_b)
    pltpu.async_copy(outv.at[pl.ds(0, _OB)], out_h.at[myidx_a], sem).wait()
    pltpu.async_copy(outv.at[pl.ds(_OB, _OB)], out_h.at[myidx_b], sem).wait()


_nms = functools.partial(
    pl.kernel,
    out_type=jax.ShapeDtypeStruct((_NPAD,), jnp.float32),
    mesh=plsc.VectorSubcoreMesh(core_axis_name="c", subcore_axis_name="s"),
    compiler_params=pltpu.CompilerParams(needs_layout_passes=False),
    scratch_types=[
        pltpu.VMEM((_NPAD,), jnp.float32),
        pltpu.VMEM((_NPAD,), jnp.float32),
        pltpu.VMEM((_NPAD,), jnp.float32),
        pltpu.VMEM((_NPAD,), jnp.float32),
        pltpu.VMEM((_NPAD,), jnp.float32),
        pltpu.VMEM((_NPAD,), jnp.int32),
        pltpu.VMEM((_NPAD,), jnp.int32),
        pltpu.VMEM((_NPAD,), jnp.float32),
        pltpu.VMEM((_NPAD,), jnp.float32),
        pltpu.VMEM((_NPAD,), jnp.float32),
        pltpu.VMEM((_NPAD,), jnp.float32),
        pltpu.VMEM((_NPAD,), jnp.float32),
        pltpu.VMEM((_NPAD,), jnp.int32),
        pltpu.VMEM((_NPAD,), jnp.float32),
        pltpu.VMEM((_NW,), jnp.int32),
        pltpu.VMEM((_NW,), jnp.int32),
        pltpu.VMEM((_RPW,), jnp.float32),
        pltpu.VMEM((_OB,), jnp.int32),
        pltpu.VMEM((_OB,), jnp.int32),
        pltpu.SemaphoreType.DMA,
    ],
)(_nms_body)


def kernel(boxes, scores, pred_labels):
    xl = boxes[:, 0]
    yl = boxes[:, 1]
    xh = boxes[:, 2]
    yh = boxes[:, 3]
    wmax = jnp.max(xh - xl)

    iota = jnp.arange(_N, dtype=jnp.int32)
    xls, order = lax.sort((xl, iota), num_keys=1, is_stable=False)

    firsts = xls[0::_RPW]
    lasts = jnp.concatenate([xls[_RPW - 1::_RPW], xls[_N - 1:]])
    lo_arr = jnp.searchsorted(xls, firsts - wmax, side="left").astype(jnp.int32)
    hi_arr = jnp.searchsorted(xls, lasts + wmax, side="right").astype(jnp.int32)

    npadf = jnp.zeros((_NPAD - _N,), jnp.float32)
    xl_p = jnp.concatenate([xl, jnp.full((_NPAD - _N,), _BIG, jnp.float32)])
    yl_p = jnp.concatenate([yl, npadf])
    xh_p = jnp.concatenate([xh, npadf])
    yh_p = jnp.concatenate([yh, npadf])
    sc_p = jnp.concatenate([scores, npadf])
    lab_p = jnp.concatenate([pred_labels.astype(jnp.int32),
                             jnp.full((_NPAD - _N,), -1, jnp.int32)])
    ord_p = jnp.concatenate([order,
                             jnp.arange(_N, _NPAD, dtype=jnp.int32)])

    out = _nms(xl_p, yl_p, xh_p, yh_p, sc_p, lab_p,
               ord_p, ord_p.reshape(_NW, _OR, _OB), lo_arr, hi_arr)
    return out[:_N]
